# baseline (speedup 1.0000x reference)
import functools

import numpy as np
import jax
import jax.numpy as jnp
from jax import lax
from jax.experimental import pallas as pl
from jax.experimental.pallas import tpu as pltpu
from jax.experimental.pallas import tpu_sc as plsc

_NB = 4
_NPOINTS = 16
_CLIP = 20.0


def _perm_tables():
    a = np.zeros([_NPOINTS, _NB], dtype=np.int32)
    for i in range(_NPOINTS):
        a[i, :] = np.array(list(np.binary_repr(i, _NB)), dtype=np.int32)
    c0 = np.zeros([_NPOINTS // 2, _NB], np.int32)
    c1 = np.zeros([_NPOINTS // 2, _NB], np.int32)
    for i in range(_NB):
        c0[:, i] = np.where(a[:, i] == 0)[0]
        c1[:, i] = np.where(a[:, i] == 1)[0]
    return c0.reshape(-1), c1.reshape(-1)


_G0, _G1 = _perm_tables()
_NSHADOW = 4


@functools.cache
def _build_sc_kernel(B, K, S):
    info = plsc.get_sparse_core_info()
    NC, NS = info.num_cores, info.num_subcores
    NW = NC * NS
    L = 16
    assert B % (NW * L) == 0
    rows_w = B // NW
    groups = rows_w // L
    KS = K * S
    OUT_W = (_NPOINTS // 2) * _NB

    mesh = plsc.VectorSubcoreMesh(core_axis_name="c", subcore_axis_name="s")

    @functools.partial(
        pl.kernel,
        out_type=jax.ShapeDtypeStruct((B * OUT_W,), jnp.float32),
        mesh=mesh,
        compiler_params=pltpu.CompilerParams(needs_layout_passes=False),
        scratch_types=[
            pltpu.VMEM((rows_w * KS,), jnp.int32),
            pltpu.VMEM((rows_w * K,), jnp.float32),
            pltpu.VMEM((rows_w * OUT_W,), jnp.float32),
            pltpu.VMEM((_NSHADOW * L * _NPOINTS,), jnp.float32),
        ],
    )
    def sc_kernel(pi_hbm, d_hbm, out_hbm, pi_v, d_v, out_v, m_v):
        wid = lax.axis_index("s") * NC + lax.axis_index("c")
        base = wid * rows_w
        pltpu.sync_copy(pi_hbm.at[pl.ds(base * KS, rows_w * KS)], pi_v)
        pltpu.sync_copy(d_hbm.at[pl.ds(base * K, rows_w * K)], d_v)

        iota = lax.iota(jnp.int32, L)
        i_ks = iota * KS
        i_k = iota * K
        i_np = iota * _NPOINTS
        inf16 = jnp.full((L,), jnp.inf, jnp.float32)

        def perm(j):
            p = 3 - (iota & 3)
            g0 = ((j >> p) << (p + 1)) | (j & ((1 << p) - 1))
            return g0, g0 | (1 << p)

        g0lo, g1lo = perm(iota >> 2)
        g0hi, g1hi = perm((iota >> 2) + 4)

        def group_body(g, carry):
            for t in range(_NSHADOW * L):
                m_v[pl.ds(t * _NPOINTS, _NPOINTS)] = inf16

            def k_body(k, carry2):
                dval = plsc.load_gather(d_v, [i_k + (g * (L * K) + k)]) * 0.5
                for s in range(S):
                    pidx = i_ks + (g * (L * KS) + k * S + s)
                    pival = plsc.load_gather(pi_v, [pidx])
                    midx = i_np + (pival + (s % _NSHADOW) * (L * _NPOINTS))
                    cur = plsc.load_gather(m_v, [midx])
                    plsc.store_scatter(m_v, [midx], jnp.minimum(cur, dval))
                return carry2

            lax.fori_loop(0, K, k_body, 0, unroll=2)

            for t in range(L):
                acc = m_v[pl.ds(t * _NPOINTS, _NPOINTS)]
                for u in range(1, _NSHADOW):
                    acc = jnp.minimum(
                        acc, m_v[pl.ds((u * L + t) * _NPOINTS, _NPOINTS)])
                m_v[pl.ds(t * _NPOINTS, _NPOINTS)] = acc

            for t in range(L):
                rbase = t * _NPOINTS
                l0lo = plsc.load_gather(m_v, [g0lo + rbase])
                l1lo = plsc.load_gather(m_v, [g1lo + rbase])
                l0hi = plsc.load_gather(m_v, [g0hi + rbase])
                l1hi = plsc.load_gather(m_v, [g1hi + rbase])
                lo = jnp.clip(l0lo - l1lo, -_CLIP, _CLIP)
                hi = jnp.clip(l0hi - l1hi, -_CLIP, _CLIP)
                obase = (g * L + t) * OUT_W
                out_v[pl.ds(obase, L)] = lo
                out_v[pl.ds(obase + L, L)] = hi
            return carry

        lax.fori_loop(0, groups, group_body, 0)
        pltpu.sync_copy(out_v, out_hbm.at[pl.ds(base * OUT_W, rows_w * OUT_W)])

    return sc_kernel


def kernel(y, r, dists, path_inds, path_syms):
    B, K = dists.shape
    S = path_inds.shape[2]
    out_flat = _build_sc_kernel(B, K, S)(
        path_inds.reshape(-1), dists.reshape(-1))
    return out_flat.reshape(B, _NPOINTS // 2, _NB)


# speedup vs baseline: 1.1698x; 1.1698x over previous
import functools

import numpy as np
import jax
import jax.numpy as jnp
from jax import lax
from jax.experimental import pallas as pl
from jax.experimental.pallas import tpu as pltpu
from jax.experimental.pallas import tpu_sc as plsc

_NB = 4
_NPOINTS = 16
_CLIP = 20.0


def _perm_tables():
    a = np.zeros([_NPOINTS, _NB], dtype=np.int32)
    for i in range(_NPOINTS):
        a[i, :] = np.array(list(np.binary_repr(i, _NB)), dtype=np.int32)
    c0 = np.zeros([_NPOINTS // 2, _NB], np.int32)
    c1 = np.zeros([_NPOINTS // 2, _NB], np.int32)
    for i in range(_NB):
        c0[:, i] = np.where(a[:, i] == 0)[0]
        c1[:, i] = np.where(a[:, i] == 1)[0]
    return c0.reshape(-1), c1.reshape(-1)


_G0, _G1 = _perm_tables()
_NSHADOW = 4


@functools.cache
def _build_sc_kernel(B, K, S):
    info = plsc.get_sparse_core_info()
    NC, NS = info.num_cores, info.num_subcores
    NW = NC * NS
    L = 16
    assert B % (NW * L) == 0
    rows_w = B // NW
    groups = rows_w // L
    KS = K * S
    OUT_W = (_NPOINTS // 2) * _NB

    mesh = plsc.VectorSubcoreMesh(core_axis_name="c", subcore_axis_name="s")

    @functools.partial(
        pl.kernel,
        out_type=jax.ShapeDtypeStruct((B * OUT_W,), jnp.float32),
        mesh=mesh,
        compiler_params=pltpu.CompilerParams(needs_layout_passes=False),
        scratch_types=[
            pltpu.VMEM((rows_w * KS,), jnp.int32),
            pltpu.VMEM((rows_w * K,), jnp.float32),
            pltpu.VMEM((rows_w * OUT_W,), jnp.float32),
        ],
    )
    def sc_kernel(pi_hbm, d_hbm, out_hbm, pi_v, d_v, out_v):
        wid = lax.axis_index("s") * NC + lax.axis_index("c")
        base = wid * rows_w
        pltpu.sync_copy(pi_hbm.at[pl.ds(base * KS, rows_w * KS)], pi_v)
        pltpu.sync_copy(d_hbm.at[pl.ds(base * K, rows_w * K)], d_v)

        iota = lax.iota(jnp.int32, L)
        i_ks = iota * KS
        i_k = iota * K
        i_ow = iota * OUT_W
        one = jnp.full((L,), 1, jnp.int32)
        inf16 = jnp.full((L,), jnp.inf, jnp.float32)

        def group_body(g, carry):
            def k_body(k, maccs):
                kbase = g * (L * KS) + k * S
                bits = None
                for s in range(S):
                    pival = plsc.load_gather(pi_v, [i_ks + (kbase + s)])
                    bit = jnp.left_shift(one, pival)
                    bits = bit if bits is None else (bits | bit)
                dval = plsc.load_gather(d_v, [i_k + (g * (L * K) + k)]) * 0.5
                return tuple(
                    jnp.minimum(
                        maccs[v],
                        jnp.where((bits & (1 << v)) != 0, dval, inf16))
                    for v in range(_NPOINTS))

            maccs = lax.fori_loop(0, K, k_body, (inf16,) * _NPOINTS,
                                  unroll=2)

            for t in range(OUT_W):
                llr = jnp.clip(maccs[_G0[t]] - maccs[_G1[t]], -_CLIP, _CLIP)
                plsc.store_scatter(
                    out_v, [i_ow + (g * (L * OUT_W) + t)], llr)
            return carry

        lax.fori_loop(0, groups, group_body, 0)
        pltpu.sync_copy(out_v, out_hbm.at[pl.ds(base * OUT_W, rows_w * OUT_W)])

    return sc_kernel


def kernel(y, r, dists, path_inds, path_syms):
    B, K = dists.shape
    S = path_inds.shape[2]
    out_flat = _build_sc_kernel(B, K, S)(
        path_inds.reshape(-1), dists.reshape(-1))
    return out_flat.reshape(B, _NPOINTS // 2, _NB)
